# Initial kernel scaffold; baseline (speedup 1.0000x reference)
import jax
import jax.numpy as jnp
from jax.experimental import pallas as pl


def kernel(x, edge_index, batch, W1, b1, gam1, bet1, W2, b2, gam2, bet2, W3, b3, gam3, bet3, W4, b4, gam4, bet4, W5, b5, gam5, bet5, Wg, bg, Wf2, bf2, Wf3, bf3, Wf4, bf4):
    raise NotImplementedError("write your pallas kernel here")


# speedup vs baseline: 3.8944x; 3.8944x over previous
import functools

import jax
import jax.numpy as jnp
from jax import lax
from jax.experimental import pallas as pl
from jax.experimental.pallas import tpu as pltpu
from jax.experimental.pallas import tpu_sc as plsc

N = 10000
NP = 10240
E = 320000
EP = 327680
G = 256
BM = 1024
NTILE = 32
NBLK = 80
ROWS_PER_TILE = NP // NTILE

def _mesh():
    return plsc.VectorSubcoreMesh(core_axis_name="c", subcore_axis_name="s",
                                  num_cores=2)


_HI = jax.lax.Precision.HIGHEST


def _sc_count(dst3, zeros16, ones16):
    @functools.partial(
        pl.kernel,
        out_type=jax.ShapeDtypeStruct((2, NP, 16), jnp.float32),
        mesh=_mesh(),
        scratch_types=[
            pltpu.VMEM((128,), jnp.int32),
            pltpu.VMEM((128, 16), jnp.float32),
            pltpu.VMEM_SHARED((NP, 16), jnp.float32),
        ],
    )
    def k(dst_h, zeros_h, ones_h, out_h, idx_d, ones_v, slab):
        c = lax.axis_index("c")
        s = lax.axis_index("s")
        tid = c * 16 + s
        r0 = s * 640
        pltpu.sync_copy(zeros_h, slab.at[pl.ds(r0, 640)])
        pltpu.sync_copy(ones_h, ones_v)
        plsc.subcore_barrier()

        def body(j, carry):
            pltpu.sync_copy(dst_h.at[tid, j], idx_d)
            pltpu.sync_copy(ones_v, slab.at[idx_d], add=True)
            return carry

        lax.fori_loop(0, NBLK, body, 0)
        plsc.subcore_barrier()
        pltpu.sync_copy(slab.at[pl.ds(r0, 640)], out_h.at[c, pl.ds(r0, 640)])

    return k(dst3, zeros16, ones16)


def _sc_prop(h_chunks, src3, dst3, zeros128):
    C = len(h_chunks)

    @functools.partial(
        pl.kernel,
        out_type=jax.ShapeDtypeStruct((2, C, NP, 128), jnp.float32),
        mesh=_mesh(),
        scratch_types=[
            pltpu.VMEM((128,), jnp.int32),
            pltpu.VMEM((128,), jnp.int32),
            pltpu.VMEM((128, 128), jnp.float32),
            pltpu.VMEM_SHARED((NP, 128), jnp.float32),
            pltpu.SemaphoreType.DMA,
        ],
    )
    def k(*refs):
        h_hs = refs[:C]
        src_h, dst_h, zeros_h, out_h = refs[C:C + 4]
        idx_s, idx_d, rows, slab, sem = refs[C + 4:]
        c = lax.axis_index("c")
        s = lax.axis_index("s")
        tid = c * 16 + s
        r0 = s * 640
        for cc in range(C):
            pltpu.sync_copy(zeros_h, slab.at[pl.ds(r0, 640)])
            plsc.subcore_barrier()

            def body(j, carry):
                pltpu.sync_copy(src_h.at[tid, j], idx_s)
                pltpu.sync_copy(dst_h.at[tid, j], idx_d)
                pltpu.async_copy(h_hs[cc].at[idx_s], rows, sem).wait()
                pltpu.sync_copy(rows, slab.at[idx_d], add=True)
                return carry

            lax.fori_loop(0, NBLK, body, 0)
            plsc.subcore_barrier()
            pltpu.sync_copy(slab.at[pl.ds(r0, 640)],
                            out_h.at[c, cc, pl.ds(r0, 640)])
            plsc.subcore_barrier()

    return k(*h_chunks, src3, dst3, zeros128)


def _bn_coeffs(stats, gam, bet):
    mean = stats[0:1, :] * (1.0 / N)
    ex2 = stats[1:2, :] * (1.0 / N)
    var = ex2 - mean * mean
    sc = gam * lax.rsqrt(var + 1e-5)
    sh = bet - mean * sc
    return sc, sh


def _k_dinv(c0, c1):
    def body(c0_r, c1_r, out_r):
        d = 1.0 + c0_r[:, 0:1] + c1_r[:, 0:1]
        out_r[...] = jnp.broadcast_to(lax.rsqrt(d), (NP, 128))

    return pl.pallas_call(
        body,
        out_shape=jax.ShapeDtypeStruct((NP, 128), jnp.float32),
    )(c0, c1)


def _k_scale_chunk(xp, dinv):
    def body(x_r, d_r, o_r):
        o_r[...] = (x_r[...] * d_r[...])[None]

    ni = NP // BM
    return pl.pallas_call(
        body,
        grid=(ni,),
        in_specs=[
            pl.BlockSpec((BM, 128), lambda i: (i, 0)),
            pl.BlockSpec((BM, 128), lambda i: (i, 0)),
        ],
        out_specs=pl.BlockSpec((1, BM, 128), lambda i: (0, i, 0)),
        out_shape=jax.ShapeDtypeStruct((1, NP, 128), jnp.float32),
    )(xp, dinv)


def _k_bn_scale_chunk(U, stats, gam, bet, dinv, C):
    def body(u_r, st_r, g_r, b_r, d_r, o_r):
        sc, sh = _bn_coeffs(st_r[...], g_r[...], b_r[...])
        h = u_r[...] * sc + sh
        o_r[...] = (h * d_r[...])[None]

    ni = NP // BM
    return pl.pallas_call(
        body,
        grid=(C, ni),
        in_specs=[
            pl.BlockSpec((BM, 128), lambda j, i: (i, j)),
            pl.BlockSpec((8, 128), lambda j, i: (0, j)),
            pl.BlockSpec((1, 128), lambda j, i: (0, j)),
            pl.BlockSpec((1, 128), lambda j, i: (0, j)),
            pl.BlockSpec((BM, 128), lambda j, i: (i, 0)),
        ],
        out_specs=pl.BlockSpec((1, BM, 128), lambda j, i: (j, i, 0)),
        out_shape=jax.ShapeDtypeStruct((C, NP, 128), jnp.float32),
    )(U, stats, gam, bet, dinv)


def _k_bn_mm_scale_chunk(U, stats, gam, bet, W, dinv, C, BK):
    K = U.shape[1]
    nk = K // BK
    ni = NP // BM

    def body(u_r, st_r, g_r, b_r, w_r, d_r, o_r, acc):
        kk = pl.program_id(2)
        sc, sh = _bn_coeffs(st_r[...], g_r[...], b_r[...])
        a = u_r[...] * sc + sh
        part = jnp.dot(a, w_r[...], precision=_HI,
                       preferred_element_type=jnp.float32)

        @pl.when(kk == 0)
        def _():
            acc[...] = part

        @pl.when(kk > 0)
        def _():
            acc[...] = acc[...] + part

        @pl.when(kk == nk - 1)
        def _():
            o_r[...] = (acc[...] * d_r[...])[None]

    return pl.pallas_call(
        body,
        grid=(C, ni, nk),
        in_specs=[
            pl.BlockSpec((BM, BK), lambda j, i, kk: (i, kk)),
            pl.BlockSpec((8, BK), lambda j, i, kk: (0, kk)),
            pl.BlockSpec((1, BK), lambda j, i, kk: (0, kk)),
            pl.BlockSpec((1, BK), lambda j, i, kk: (0, kk)),
            pl.BlockSpec((BK, 128), lambda j, i, kk: (kk, j)),
            pl.BlockSpec((BM, 128), lambda j, i, kk: (i, 0)),
        ],
        out_specs=pl.BlockSpec((1, BM, 128), lambda j, i, kk: (j, i, 0)),
        out_shape=jax.ShapeDtypeStruct((C, NP, 128), jnp.float32),
        scratch_shapes=[pltpu.VMEM((BM, 128), jnp.float32)],
    )(U, stats, gam, bet, W, dinv)


def _row_mask(i, bm):
    row = i * bm + lax.broadcasted_iota(jnp.int32, (bm, 1), 0)
    return (row < N).astype(jnp.float32)


def _k_combine_mm_relu_stats(p0, p1, q, dinv, W, bias, BN_):
    C = q.shape[0]
    K = C * 128
    F = W.shape[1]
    nj = F // BN_
    ni = NP // BM

    def body(p0_r, p1_r, q_r, d_r, w_r, b_r, u_r, st_r):
        j = pl.program_id(0)
        i = pl.program_id(1)
        d = d_r[...]
        a = jnp.concatenate(
            [d * (p0_r[c] + p1_r[c] + q_r[c]) for c in range(C)], axis=1)
        u = jnp.maximum(jnp.dot(a, w_r[...], precision=_HI,
                                preferred_element_type=jnp.float32)
                        + b_r[...], 0.0)
        u_r[...] = u
        m = _row_mask(i, BM)
        um = u * m
        su = jnp.sum(um, axis=0, keepdims=True)
        sq = jnp.sum(um * u, axis=0, keepdims=True)
        z = jnp.zeros((6, BN_), jnp.float32)
        part = jnp.concatenate([su, sq, z], axis=0)

        @pl.when(i == 0)
        def _():
            st_r[...] = part

        @pl.when(i > 0)
        def _():
            st_r[...] = st_r[...] + part

    return pl.pallas_call(
        body,
        grid=(nj, ni),
        in_specs=[
            pl.BlockSpec((C, BM, 128), lambda j, i: (0, i, 0)),
            pl.BlockSpec((C, BM, 128), lambda j, i: (0, i, 0)),
            pl.BlockSpec((C, BM, 128), lambda j, i: (0, i, 0)),
            pl.BlockSpec((BM, 128), lambda j, i: (i, 0)),
            pl.BlockSpec((K, BN_), lambda j, i: (0, j)),
            pl.BlockSpec((1, BN_), lambda j, i: (0, j)),
        ],
        out_specs=[
            pl.BlockSpec((BM, BN_), lambda j, i: (i, j)),
            pl.BlockSpec((8, BN_), lambda j, i: (0, j)),
        ],
        out_shape=[
            jax.ShapeDtypeStruct((NP, F), jnp.float32),
            jax.ShapeDtypeStruct((8, F), jnp.float32),
        ],
    )(p0, p1, q, dinv, W, bias)


def _k_combine_relu_stats(p0, p1, z, dinv, bias):
    C = z.shape[0]
    ni = NP // BM

    def body(p0_r, p1_r, z_r, d_r, b_r, u_r, st_r):
        i = pl.program_id(1)
        u = jnp.maximum(d_r[...] * (p0_r[0] + p1_r[0] + z_r[0]) + b_r[...],
                        0.0)
        u_r[...] = u
        m = _row_mask(i, BM)
        um = u * m
        su = jnp.sum(um, axis=0, keepdims=True)
        sq = jnp.sum(um * u, axis=0, keepdims=True)
        part = jnp.concatenate(
            [su, sq, jnp.zeros((6, 128), jnp.float32)], axis=0)

        @pl.when(i == 0)
        def _():
            st_r[...] = part

        @pl.when(i > 0)
        def _():
            st_r[...] = st_r[...] + part

    return pl.pallas_call(
        body,
        grid=(C, ni),
        in_specs=[
            pl.BlockSpec((1, BM, 128), lambda j, i: (j, i, 0)),
            pl.BlockSpec((1, BM, 128), lambda j, i: (j, i, 0)),
            pl.BlockSpec((1, BM, 128), lambda j, i: (j, i, 0)),
            pl.BlockSpec((BM, 128), lambda j, i: (i, 0)),
            pl.BlockSpec((1, 128), lambda j, i: (0, j)),
        ],
        out_specs=[
            pl.BlockSpec((BM, 128), lambda j, i: (i, j)),
            pl.BlockSpec((8, 128), lambda j, i: (0, j)),
        ],
        out_shape=[
            jax.ShapeDtypeStruct((NP, C * 128), jnp.float32),
            jax.ShapeDtypeStruct((8, C * 128), jnp.float32),
        ],
    )(p0, p1, z, dinv, bias)


def _k_gate(U, stats, gam, bet, WgP, bgP):
    K = U.shape[1]
    BK = 512
    nk = K // BK
    ni = NP // BM

    def body(u_r, st_r, g_r, b_r, w_r, bg_r, o_r, acc):
        kk = pl.program_id(1)
        sc, sh = _bn_coeffs(st_r[...], g_r[...], b_r[...])
        a = u_r[...] * sc + sh
        part = jnp.dot(a, w_r[...], precision=_HI,
                       preferred_element_type=jnp.float32)

        @pl.when(kk == 0)
        def _():
            acc[...] = part

        @pl.when(kk > 0)
        def _():
            acc[...] = acc[...] + part

        @pl.when(kk == nk - 1)
        def _():
            o_r[...] = acc[...] + bg_r[...]

    return pl.pallas_call(
        body,
        grid=(ni, nk),
        in_specs=[
            pl.BlockSpec((BM, BK), lambda i, kk: (i, kk)),
            pl.BlockSpec((8, BK), lambda i, kk: (0, kk)),
            pl.BlockSpec((1, BK), lambda i, kk: (0, kk)),
            pl.BlockSpec((1, BK), lambda i, kk: (0, kk)),
            pl.BlockSpec((BK, 128), lambda i, kk: (kk, 0)),
            pl.BlockSpec((1, 128), lambda i, kk: (0, 0)),
        ],
        out_specs=pl.BlockSpec((BM, 128), lambda i, kk: (i, 0)),
        out_shape=jax.ShapeDtypeStruct((NP, 128), jnp.float32),
        scratch_shapes=[pltpu.VMEM((BM, 128), jnp.float32)],
    )(U, stats, gam, bet, WgP, bgP)


def _k_gmax(gcol, batchcol):
    ni = NP // BM

    def body(g_r, b_r, o_r):
        i = pl.program_id(0)
        gate = g_r[:, 0:1]
        seg = b_r[:, 0:1]
        ids = lax.broadcasted_iota(jnp.int32, (1, G), 1)
        m = jnp.where(seg == ids, gate, -1e30)
        red = jnp.max(m, axis=0, keepdims=True)
        red8 = jnp.broadcast_to(red, (8, G))

        @pl.when(i == 0)
        def _():
            o_r[...] = red8

        @pl.when(i > 0)
        def _():
            o_r[...] = jnp.maximum(o_r[...], red8)

    return pl.pallas_call(
        body,
        grid=(ni,),
        in_specs=[
            pl.BlockSpec((BM, 128), lambda i: (i, 0)),
            pl.BlockSpec((BM, 128), lambda i: (i, 0)),
        ],
        out_specs=pl.BlockSpec((8, G), lambda i: (0, 0)),
        out_shape=jax.ShapeDtypeStruct((8, G), jnp.float32),
    )(gcol, batchcol)


def _k_pool(U, stats, gam, bet, gcol, batchcol, gmax):
    F = U.shape[1]
    ni = NP // BM

    def body(u_r, st_r, g_r, b_r, gc_r, bc_r, gm_r, o_r):
        i = pl.program_id(0)
        sc, sh = _bn_coeffs(st_r[...], g_r[...], b_r[...])
        h = u_r[...] * sc + sh
        aug = jnp.concatenate([h, jnp.ones((BM, 128), jnp.float32)], axis=1)
        gate = gc_r[:, 0:1]
        seg = bc_r[:, 0:1]
        ids = lax.broadcasted_iota(jnp.int32, (1, G), 1)
        gm = gm_r[0:1, :]
        lg = jnp.where(seg == ids, gate - gm, -1e30)
        w = jnp.exp(lg)
        part = lax.dot_general(w, aug, (((0,), (0,)), ((), ())),
                               precision=_HI,
                               preferred_element_type=jnp.float32)

        @pl.when(i == 0)
        def _():
            o_r[...] = part

        @pl.when(i > 0)
        def _():
            o_r[...] = o_r[...] + part

    return pl.pallas_call(
        body,
        grid=(ni,),
        in_specs=[
            pl.BlockSpec((BM, F), lambda i: (i, 0)),
            pl.BlockSpec((8, F), lambda i: (0, 0)),
            pl.BlockSpec((1, F), lambda i: (0, 0)),
            pl.BlockSpec((1, F), lambda i: (0, 0)),
            pl.BlockSpec((BM, 128), lambda i: (i, 0)),
            pl.BlockSpec((BM, 128), lambda i: (i, 0)),
            pl.BlockSpec((8, G), lambda i: (0, 0)),
        ],
        out_specs=pl.BlockSpec((G, F + 128), lambda i: (0, 0)),
        out_shape=jax.ShapeDtypeStruct((G, F + 128), jnp.float32),
    )(U, stats, gam, bet, gcol, batchcol, gmax)


def _k_head(pooledaug, Wf2, bf2, Wf3p, bf3p, Wf4p, bf4p):
    def body(p_r, w2_r, b2_r, w3_r, b3_r, w4_r, b4_r, o_r):
        pa = p_r[...]
        denom = jnp.maximum(pa[:, 1024:1025], 1e-16)
        P = pa[:, :1024] / denom
        o = jnp.maximum(jnp.dot(P, w2_r[...], precision=_HI,
                                preferred_element_type=jnp.float32)
                        + b2_r[...], 0.0)
        o = jnp.maximum(jnp.dot(o, w3_r[...], precision=_HI,
                                preferred_element_type=jnp.float32)
                        + b3_r[...], 0.0)
        o_r[...] = jnp.dot(o, w4_r[...], precision=_HI,
                           preferred_element_type=jnp.float32) + b4_r[...]

    return pl.pallas_call(
        body,
        out_shape=jax.ShapeDtypeStruct((G, 128), jnp.float32),
    )(pooledaug, Wf2, bf2, Wf3p, bf3p, Wf4p, bf4p)


def kernel(x, edge_index, batch,
           W1, b1, gam1, bet1, W2, b2, gam2, bet2, W3, b3, gam3, bet3,
           W4, b4, gam4, bet4, W5, b5, gam5, bet5,
           Wg, bg, Wf2, bf2, Wf3, bf3, Wf4, bf4):
    f32 = jnp.float32
    xp = jnp.pad(x, ((0, NP - N), (0, 0)))
    src3 = jnp.pad(edge_index[0], (0, EP - E), constant_values=NP - 1)
    src3 = src3.reshape(NTILE, NBLK, 128)
    dst3 = jnp.pad(edge_index[1], (0, EP - E), constant_values=NP - 1)
    dst3 = dst3.reshape(NTILE, NBLK, 128)
    batchcol = jnp.broadcast_to(
        jnp.pad(batch, (0, NP - N), constant_values=G)[:, None], (NP, 128))
    zeros16 = jnp.zeros((640, 16), f32)
    ones16 = jnp.ones((128, 16), f32)
    zeros128 = jnp.zeros((640, 128), f32)

    r = lambda v: v.reshape(1, -1)
    b1r, b2r, b3r, b4r, b5r = r(b1), r(b2), r(b3), r(b4), r(b5)
    g1r, g2r, g3r, g4r, g5r = r(gam1), r(gam2), r(gam3), r(gam4), r(gam5)
    t1r, t2r, t3r, t4r, t5r = r(bet1), r(bet2), r(bet3), r(bet4), r(bet5)
    WgP = jnp.pad(Wg, ((0, 0), (0, 127)))
    bgP = jnp.pad(r(bg), ((0, 0), (0, 127)))
    Wf3p = jnp.pad(Wf3, ((0, 0), (0, 112)))
    bf3p = jnp.pad(r(bf3), ((0, 0), (0, 112)))
    Wf4p = jnp.pad(Wf4, ((0, 112), (0, 127)))
    bf4p = jnp.pad(r(bf4), ((0, 0), (0, 127)))
    bf2r = r(bf2)

    counts = _sc_count(dst3, zeros16, ones16)
    dinv = _k_dinv(counts[0], counts[1])

    xq = _k_scale_chunk(xp, dinv)
    p = _sc_prop([xq[0]], src3, dst3, zeros128)
    U1, st1 = _k_combine_mm_relu_stats(p[0], p[1], xq, dinv, W1, b1r, 512)

    z2 = _k_bn_mm_scale_chunk(U1, st1, g1r, t1r, W2, dinv, 4, 512)
    p = _sc_prop([z2[0], z2[1], z2[2], z2[3]], src3, dst3, zeros128)
    U2, st2 = _k_combine_relu_stats(p[0], p[1], z2, dinv, b2r)

    z3 = _k_bn_mm_scale_chunk(U2, st2, g2r, t2r, W3, dinv, 2, 512)
    p = _sc_prop([z3[0], z3[1]], src3, dst3, zeros128)
    U3, st3 = _k_combine_relu_stats(p[0], p[1], z3, dinv, b3r)

    q4 = _k_bn_scale_chunk(U3, st3, g3r, t3r, dinv, 2)
    p = _sc_prop([q4[0], q4[1]], src3, dst3, zeros128)
    U4, st4 = _k_combine_mm_relu_stats(p[0], p[1], q4, dinv, W4, b4r, 512)

    q5 = _k_bn_scale_chunk(U4, st4, g4r, t4r, dinv, 4)
    p = _sc_prop([q5[0], q5[1], q5[2], q5[3]], src3, dst3, zeros128)
    U5, st5 = _k_combine_mm_relu_stats(p[0], p[1], q5, dinv, W5, b5r, 512)

    gcol = _k_gate(U5, st5, g5r, t5r, WgP, bgP)
    gmax = _k_gmax(gcol, batchcol)
    pooledaug = _k_pool(U5, st5, g5r, t5r, gcol, batchcol, gmax)
    out = _k_head(pooledaug, Wf2, bf2r, Wf3p, bf3p, Wf4p, bf4p)
    return out[:, :1]
